# Initial kernel scaffold; baseline (speedup 1.0000x reference)
import jax
import jax.numpy as jnp
from jax.experimental import pallas as pl


def kernel(input1, input2):
    raise NotImplementedError("write your pallas kernel here")


# speedup vs baseline: 2.0017x; 2.0017x over previous
import jax
import jax.numpy as jnp
from jax.experimental import pallas as pl
from jax.experimental.pallas import tpu as pltpu

N = 2048
T = 256
NT = N // T


def _emd_body(x1a, x1b, x1c, x2a, x2b, x2c, out_ref, d2_ref, e_ref, s_ref, a_ref):
    b1 = x2a[0]
    b2 = x2b[0]
    b3 = x2c[0]

    def build_d2(t, carry):
        rs = pl.ds(t * T, T)
        p1 = x1a[0, rs, :]
        p2 = x1b[0, rs, :]
        p3 = x1c[0, rs, :]
        d2_ref[rs, :] = (p1 - b1) ** 2 + (p2 - b2) ** 2 + (p3 - b3) ** 2
        return carry

    jax.lax.fori_loop(0, NT, build_d2, 0)

    satl = jnp.ones((N, 1), dtype=jnp.float32)
    satr = jnp.ones((1, N), dtype=jnp.float32)
    cost = jnp.zeros((1, 1), dtype=jnp.float32)

    for j in range(8, -3, -1):
        level = 0.0 if j == -2 else -(4.0 ** j)
        satr_c = jnp.transpose(satr)

        def pass_a(t, carry):
            rs = pl.ds(t * T, T)
            e_t = jnp.exp(level * d2_ref[rs, :])
            e_ref[rs, :] = e_t
            s_ref[rs, :] = jax.lax.dot(
                e_t, satr_c, preferred_element_type=jnp.float32
            )
            return carry

        jax.lax.fori_loop(0, NT, pass_a, 0)

        s = s_ref[:, :] + 1e-9
        a_ref[:, :] = satl / s

        def pass_b(t, cs):
            rs = pl.ds(t * T, T)
            a_t = a_ref[rs, :]
            return cs + jnp.sum(e_ref[rs, :] * a_t, axis=0, keepdims=True)

        cs = jax.lax.fori_loop(
            0, NT, pass_b, jnp.zeros((1, N), dtype=jnp.float32)
        )
        ssr = satr * cs
        r = jnp.minimum(satr / (ssr + 1e-9), 1.0)
        u = satr * r
        u_c = jnp.transpose(u)
        satr = jnp.maximum(satr - ssr * r, 0.0)

        def pass_c(t, cost):
            rs = pl.ds(t * T, T)
            a_t = a_ref[rs, :]
            e_t = e_ref[rs, :]
            srow_t = a_t * jax.lax.dot(
                e_t, u_c, preferred_element_type=jnp.float32
            )
            s_ref[rs, :] = srow_t
            d_t = jnp.sqrt(jnp.maximum(d2_ref[rs, :], 1e-12))
            h_t = e_t * d_t
            c_t = jax.lax.dot(h_t, u_c, preferred_element_type=jnp.float32)
            return cost + jnp.sum(a_t * c_t, axis=0, keepdims=True)

        cost = jax.lax.fori_loop(0, NT, pass_c, cost)
        satl = jnp.maximum(satl - s_ref[:, :], 0.0)

    out_ref[0] = cost


def kernel(input1, input2):
    B = input1.shape[0]
    x2t = jnp.transpose(input2, (0, 2, 1))
    ins = (
        input1[:, :, 0:1],
        input1[:, :, 1:2],
        input1[:, :, 2:3],
        x2t[:, 0:1, :],
        x2t[:, 1:2, :],
        x2t[:, 2:3, :],
    )
    col_spec = pl.BlockSpec((1, N, 1), lambda b: (b, 0, 0))
    row_spec = pl.BlockSpec((1, 1, N), lambda b: (b, 0, 0))
    out = pl.pallas_call(
        _emd_body,
        grid=(B,),
        in_specs=[col_spec, col_spec, col_spec, row_spec, row_spec, row_spec],
        out_specs=pl.BlockSpec((1, 1, 1), lambda b: (b, 0, 0)),
        out_shape=jax.ShapeDtypeStruct((B, 1, 1), jnp.float32),
        scratch_shapes=[
            pltpu.VMEM((N, N), jnp.float32),
            pltpu.VMEM((N, N), jnp.float32),
            pltpu.VMEM((N, 1), jnp.float32),
            pltpu.VMEM((N, 1), jnp.float32),
        ],
    )(*ins)
    return out[:, 0, 0]


# speedup vs baseline: 2.5397x; 1.2688x over previous
import jax
import jax.numpy as jnp
from jax.experimental import pallas as pl
from jax.experimental.pallas import tpu as pltpu

N = 2048
T = 256
NT = N // T

_F32 = jnp.float32
_BF16 = jnp.bfloat16


def _emd_body(x1a, x1b, x1c, x2a, x2b, x2c, out_ref,
              d2_ref, d_ref, e0_ref, e1_ref, rc0_ref, rc1_ref, ab_ref):
    b1 = x2a[0]
    b2 = x2b[0]
    b3 = x2c[0]

    ones_col = jnp.ones((N, 1), dtype=_BF16)

    def build(t, carry):
        rs = pl.ds(t * T, T)
        p1 = x1a[0, rs, :]
        p2 = x1b[0, rs, :]
        p3 = x1c[0, rs, :]
        d2_t = (p1 - b1) ** 2 + (p2 - b2) ** 2 + (p3 - b3) ** 2
        d2_ref[rs, :] = d2_t
        d_ref[rs, :] = jnp.sqrt(jnp.maximum(d2_t, 1e-12)).astype(_BF16)
        e_t = jnp.exp((-(4.0 ** 8)) * d2_t).astype(_BF16)
        e0_ref[rs, :] = e_t
        rc0_ref[rs, :] = jax.lax.dot(e_t, ones_col,
                                     preferred_element_type=_F32)
        return carry

    jax.lax.fori_loop(0, NT, build, 0)

    satl = jnp.ones((1, N), dtype=_F32)
    satr = jnp.ones((1, N), dtype=_F32)
    cost = jnp.zeros((1, N), dtype=_F32)
    s = jnp.transpose(rc0_ref[:, :])

    for idx in range(10):
        j = 8 - idx
        cur = e0_ref if idx % 2 == 0 else e1_ref
        nxt = e1_ref if idx % 2 == 0 else e0_ref
        has_next = idx < 9
        level_next = -(4.0 ** (j - 1))

        a = satl / (s + 1e-9)
        ab_ref[:, :] = a.astype(_BF16)

        def pass_l1(t, cs):
            rs = pl.ds(t * T, T)
            if has_next:
                nxt[rs, :] = jnp.exp(level_next * d2_ref[rs, :]).astype(_BF16)
            return cs + jax.lax.dot(
                ab_ref[:, rs], cur[rs, :], preferred_element_type=_F32
            )

        cs = jax.lax.fori_loop(0, NT, pass_l1,
                               jnp.zeros((1, N), dtype=_F32))

        ssr = satr * cs
        r = jnp.minimum(satr / (ssr + 1e-9), 1.0)
        u = satr * r
        satr = jnp.maximum(satr - ssr * r, 0.0)
        u_b = u.astype(_BF16)
        satr_b = satr.astype(_BF16)

        def pass_l2(t, cost_c):
            rs = pl.ds(t * T, T)
            e_t = cur[rs, :]
            q_t = e_t * u_b
            r_t = q_t * d_ref[rs, :]
            rc0_ref[rs, :] = jax.lax.dot(q_t, ones_col,
                                         preferred_element_type=_F32)
            if has_next:
                p_t = nxt[rs, :] * satr_b
                rc1_ref[rs, :] = jax.lax.dot(p_t, ones_col,
                                             preferred_element_type=_F32)
            return cost_c + jax.lax.dot(
                ab_ref[:, rs], r_t, preferred_element_type=_F32
            )

        cost = jax.lax.fori_loop(0, NT, pass_l2, cost)

        eu = jnp.transpose(rc0_ref[:, :])
        satl = jnp.maximum(satl - a * eu, 0.0)
        if has_next:
            s = jnp.transpose(rc1_ref[:, :])

    s0 = jnp.sum(satr) + 1e-9
    lsum = jnp.sum(satl)
    ss = satr * (lsum / s0) + 1e-9
    r = jnp.minimum(satr / ss, 1.0)
    u_b = (satr * r).astype(_BF16)
    ab_ref[:, :] = (satl * (1.0 / s0)).astype(_BF16)

    def pass_final(t, cost_c):
        rs = pl.ds(t * T, T)
        r_t = d_ref[rs, :] * u_b
        return cost_c + jax.lax.dot(
            ab_ref[:, rs], r_t, preferred_element_type=_F32
        )

    cost = jax.lax.fori_loop(0, NT, pass_final, cost)

    out_ref[0] = jnp.sum(cost, axis=1, keepdims=True)


def kernel(input1, input2):
    B = input1.shape[0]
    x2t = jnp.transpose(input2, (0, 2, 1))
    ins = (
        input1[:, :, 0:1],
        input1[:, :, 1:2],
        input1[:, :, 2:3],
        x2t[:, 0:1, :],
        x2t[:, 1:2, :],
        x2t[:, 2:3, :],
    )
    col_spec = pl.BlockSpec((1, N, 1), lambda b: (b, 0, 0))
    row_spec = pl.BlockSpec((1, 1, N), lambda b: (b, 0, 0))
    out = pl.pallas_call(
        _emd_body,
        grid=(B,),
        in_specs=[col_spec, col_spec, col_spec, row_spec, row_spec, row_spec],
        out_specs=pl.BlockSpec((1, 1, 1), lambda b: (b, 0, 0)),
        out_shape=jax.ShapeDtypeStruct((B, 1, 1), jnp.float32),
        scratch_shapes=[
            pltpu.VMEM((N, N), _F32),
            pltpu.VMEM((N, N), _BF16),
            pltpu.VMEM((N, N), _BF16),
            pltpu.VMEM((N, N), _BF16),
            pltpu.VMEM((N, 1), _F32),
            pltpu.VMEM((N, 1), _F32),
            pltpu.VMEM((1, N), _BF16),
        ],
    )(*ins)
    return out[:, 0, 0]


# speedup vs baseline: 2.6594x; 1.0471x over previous
import jax
import jax.numpy as jnp
from jax.experimental import pallas as pl
from jax.experimental.pallas import tpu as pltpu

N = 2048
T = 256
NT = N // T

_F32 = jnp.float32
_BF16 = jnp.bfloat16


def _emd_body(x1a, x1b, x1c, x2a, x2b, x2c, out_ref,
              d2_ref, d_ref, e0_ref, e1_ref, rc0_ref, rc1_ref, ab_ref):
    b1 = x2a[0]
    b2 = x2b[0]
    b3 = x2c[0]

    ones_col = jnp.ones((N, 1), dtype=_BF16)

    def build(t, carry):
        rs = pl.ds(t * T, T)
        p1 = x1a[0, rs, :]
        p2 = x1b[0, rs, :]
        p3 = x1c[0, rs, :]
        d2_t = (p1 - b1) ** 2 + (p2 - b2) ** 2 + (p3 - b3) ** 2
        d2_ref[rs, :] = d2_t
        d_ref[rs, :] = jnp.sqrt(jnp.maximum(d2_t, 1e-12)).astype(_BF16)
        e_t = jnp.exp((-(4.0 ** 8)) * d2_t).astype(_BF16)
        e0_ref[rs, :] = e_t
        rc0_ref[:, rs] = jnp.transpose(
            jax.lax.dot(e_t, ones_col, preferred_element_type=_F32))
        return carry

    jax.lax.fori_loop(0, NT, build, 0)

    satl = jnp.ones((1, N), dtype=_F32)
    satr = jnp.ones((1, N), dtype=_F32)
    cost = jnp.zeros((1, N), dtype=_F32)
    s = rc0_ref[:, :]

    for idx in range(10):
        j = 8 - idx
        cur = e0_ref if idx % 2 == 0 else e1_ref
        nxt = e1_ref if idx % 2 == 0 else e0_ref
        has_next = idx < 9
        level_next = -(4.0 ** (j - 1))

        a = satl / (s + 1e-9)
        ab_ref[:, :] = a.astype(_BF16)

        def pass_l1(t, cs):
            rs = pl.ds(t * T, T)
            if has_next:
                nxt[rs, :] = jnp.exp(level_next * d2_ref[rs, :]).astype(_BF16)
            return cs + jax.lax.dot(
                ab_ref[:, rs], cur[rs, :], preferred_element_type=_F32
            )

        cs = jax.lax.fori_loop(0, NT, pass_l1,
                               jnp.zeros((1, N), dtype=_F32))

        ssr = satr * cs
        r = jnp.minimum(satr / (ssr + 1e-9), 1.0)
        u = satr * r
        satr = jnp.maximum(satr - ssr * r, 0.0)
        u_b = u.astype(_BF16)
        satr_b = satr.astype(_BF16)

        def pass_l2(t, cost_c):
            rs = pl.ds(t * T, T)
            e_t = cur[rs, :]
            q_t = e_t * u_b
            r_t = q_t * d_ref[rs, :]
            rc0_ref[:, rs] = jnp.transpose(
                jax.lax.dot(q_t, ones_col, preferred_element_type=_F32))
            if has_next:
                p_t = nxt[rs, :] * satr_b
                rc1_ref[:, rs] = jnp.transpose(
                    jax.lax.dot(p_t, ones_col, preferred_element_type=_F32))
            return cost_c + jax.lax.dot(
                ab_ref[:, rs], r_t, preferred_element_type=_F32
            )

        cost = jax.lax.fori_loop(0, NT, pass_l2, cost)

        eu = rc0_ref[:, :]
        satl = jnp.maximum(satl - a * eu, 0.0)
        if has_next:
            s = rc1_ref[:, :]

    s0 = jnp.sum(satr) + 1e-9
    lsum = jnp.sum(satl)
    ss = satr * (lsum / s0) + 1e-9
    r = jnp.minimum(satr / ss, 1.0)
    u_b = (satr * r).astype(_BF16)
    ab_ref[:, :] = (satl * (1.0 / s0)).astype(_BF16)

    def pass_final(t, cost_c):
        rs = pl.ds(t * T, T)
        r_t = d_ref[rs, :] * u_b
        return cost_c + jax.lax.dot(
            ab_ref[:, rs], r_t, preferred_element_type=_F32
        )

    cost = jax.lax.fori_loop(0, NT, pass_final, cost)

    out_ref[0] = jnp.sum(cost, axis=1, keepdims=True)


def kernel(input1, input2):
    B = input1.shape[0]
    x2t = jnp.transpose(input2, (0, 2, 1))
    ins = (
        input1[:, :, 0:1],
        input1[:, :, 1:2],
        input1[:, :, 2:3],
        x2t[:, 0:1, :],
        x2t[:, 1:2, :],
        x2t[:, 2:3, :],
    )
    col_spec = pl.BlockSpec((1, N, 1), lambda b: (b, 0, 0))
    row_spec = pl.BlockSpec((1, 1, N), lambda b: (b, 0, 0))
    out = pl.pallas_call(
        _emd_body,
        grid=(B,),
        in_specs=[col_spec, col_spec, col_spec, row_spec, row_spec, row_spec],
        out_specs=pl.BlockSpec((1, 1, 1), lambda b: (b, 0, 0)),
        out_shape=jax.ShapeDtypeStruct((B, 1, 1), jnp.float32),
        scratch_shapes=[
            pltpu.VMEM((N, N), _F32),
            pltpu.VMEM((N, N), _BF16),
            pltpu.VMEM((N, N), _BF16),
            pltpu.VMEM((N, N), _BF16),
            pltpu.VMEM((1, N), _F32),
            pltpu.VMEM((1, N), _F32),
            pltpu.VMEM((1, N), _BF16),
        ],
    )(*ins)
    return out[:, 0, 0]
